# Initial kernel scaffold; baseline (speedup 1.0000x reference)
import jax
import jax.numpy as jnp
from jax.experimental import pallas as pl


def kernel(x, edge_index, batch, gin0_W1, gin0_b1, gin0_W2, gin0_b2, gin1_W1, gin1_b1, gin1_W2, gin1_b2, gin2_W1, gin2_b1, gin2_W2, gin2_b2, lin_W, lin_b):
    raise NotImplementedError("write your pallas kernel here")


# speedup vs baseline: 4.5180x; 4.5180x over previous
import functools

import jax
import jax.numpy as jnp
from jax import lax
from jax.experimental import pallas as pl
from jax.experimental.pallas import tpu as pltpu
from jax.experimental.pallas import tpu_sc as plsc

_N = 10000
_E = 320000
_D = 128
_G = 64

_NC = 2
_NS = 16
_NW = _NC * _NS
_EPW = _E // _NW
_CH = 80
_NCHUNK = _EPW // _CH
_RPT = 624
_REM = _N - _NS * _RPT


def _make_agg_kernel():
    mesh = plsc.VectorSubcoreMesh(core_axis_name="c", subcore_axis_name="s")

    @functools.partial(
        pl.kernel,
        mesh=mesh,
        out_type=jax.ShapeDtypeStruct((_NC * _N, _D), jnp.float32),
        scratch_types=[
            pltpu.VMEM((_CH,), jnp.int32),
            pltpu.VMEM((_CH,), jnp.int32),
            pltpu.VMEM((_CH, _D), jnp.float32),
            pltpu.VMEM_SHARED((_N, _D), jnp.float32),
            pltpu.SemaphoreType.DMA,
        ],
    )
    def agg(h_hbm, src_hbm, dst_hbm, zeros_hbm, out_hbm,
            idx_s, idx_d, rows, accum, sem):
        c = lax.axis_index("c")
        s = lax.axis_index("s")
        pltpu.sync_copy(zeros_hbm.at[pl.ds(s * _RPT, _RPT)],
                        accum.at[pl.ds(s * _RPT, _RPT)])

        @pl.when(s == 0)
        def _():
            pltpu.sync_copy(zeros_hbm.at[pl.ds(_NS * _RPT, _REM)],
                            accum.at[pl.ds(_NS * _RPT, _REM)])

        plsc.subcore_barrier()

        base = (c * _NS + s) * _EPW

        def body(i, carry):
            off = base + i * _CH
            pltpu.sync_copy(src_hbm.at[pl.ds(off, _CH)], idx_s)
            pltpu.sync_copy(dst_hbm.at[pl.ds(off, _CH)], idx_d)
            pltpu.async_copy(h_hbm.at[idx_s], rows, sem).wait()
            pltpu.sync_copy(rows, accum.at[idx_d], add=True)
            return carry

        lax.fori_loop(0, _NCHUNK, body, 0)
        plsc.subcore_barrier()
        pltpu.sync_copy(accum.at[pl.ds(s * _RPT, _RPT)],
                        out_hbm.at[pl.ds(c * _N + s * _RPT, _RPT)])

        @pl.when(s == 0)
        def _():
            pltpu.sync_copy(accum.at[pl.ds(_NS * _RPT, _REM)],
                            out_hbm.at[pl.ds(c * _N + _NS * _RPT, _REM)])

    return agg


def _mlp(h, parts, W1, b1, W2, b2, relu_out):
    BN = 2000
    nblk = _N // BN

    def body(h_ref, a0_ref, a1_ref, W1_ref, b1_ref, W2_ref, b2_ref, o_ref):
        t = h_ref[...] + a0_ref[...] + a1_ref[...]
        t = jnp.dot(t, W1_ref[...], preferred_element_type=jnp.float32,
                    precision=lax.Precision.HIGHEST) + b1_ref[...]
        t = jnp.maximum(t, 0.0)
        t = jnp.dot(t, W2_ref[...], preferred_element_type=jnp.float32,
                    precision=lax.Precision.HIGHEST) + b2_ref[...]
        if relu_out:
            t = jnp.maximum(t, 0.0)
        o_ref[...] = t

    return pl.pallas_call(
        body,
        grid=(nblk,),
        in_specs=[
            pl.BlockSpec((BN, _D), lambda i: (i, 0)),
            pl.BlockSpec((BN, _D), lambda i: (i, 0)),
            pl.BlockSpec((BN, _D), lambda i: (i + nblk, 0)),
            pl.BlockSpec((_D, _D), lambda i: (0, 0)),
            pl.BlockSpec((1, _D), lambda i: (0, 0)),
            pl.BlockSpec((_D, _D), lambda i: (0, 0)),
            pl.BlockSpec((1, _D), lambda i: (0, 0)),
        ],
        out_specs=pl.BlockSpec((BN, _D), lambda i: (i, 0)),
        out_shape=jax.ShapeDtypeStruct((_N, _D), jnp.float32),
    )(h, parts, parts, W1, b1.reshape(1, _D), W2, b2.reshape(1, _D))


def _pool(h, batch2d, lin_W, lin_b):
    def body(h_ref, b_ref, W_ref, bias_ref, o_ref):
        seg = b_ref[...]
        gids = lax.broadcasted_iota(jnp.int32, (_G, _N), 0)
        mask = (seg == gids).astype(jnp.float32)
        psum = jnp.dot(mask, h_ref[...], preferred_element_type=jnp.float32,
                       precision=lax.Precision.HIGHEST)
        cnt = jnp.sum(mask, axis=1, keepdims=True)
        pooled = psum / jnp.maximum(cnt, 1.0)
        o_ref[...] = jnp.dot(pooled, W_ref[...],
                             preferred_element_type=jnp.float32,
                             precision=lax.Precision.HIGHEST) + bias_ref[...]

    return pl.pallas_call(
        body,
        out_shape=jax.ShapeDtypeStruct((_G, _D), jnp.float32),
    )(h, batch2d, lin_W, lin_b.reshape(1, _D))


def kernel(x, edge_index, batch, gin0_W1, gin0_b1, gin0_W2, gin0_b2,
           gin1_W1, gin1_b1, gin1_W2, gin1_b2,
           gin2_W1, gin2_b1, gin2_W2, gin2_b2, lin_W, lin_b):
    src = edge_index[0]
    dst = edge_index[1]
    zeros = jnp.zeros((_N, _D), jnp.float32)
    aggk = _make_agg_kernel()

    def layer(h, W1, b1, W2, b2, relu_out):
        parts = aggk(h, src, dst, zeros)
        return _mlp(h, parts, W1, b1, W2, b2, relu_out)

    h = layer(x, gin0_W1, gin0_b1, gin0_W2, gin0_b2, True)
    h = layer(h, gin1_W1, gin1_b1, gin1_W2, gin1_b2, True)
    h = layer(h, gin2_W1, gin2_b1, gin2_W2, gin2_b2, False)
    return _pool(h, batch.reshape(1, _N), lin_W, lin_b)
